# Initial kernel scaffold; baseline (speedup 1.0000x reference)
import jax
import jax.numpy as jnp
from jax.experimental import pallas as pl


def kernel(target_word, context_words, neg_words, u_table, v_table, W_dur, b_dur):
    raise NotImplementedError("write your pallas kernel here")


# speedup vs baseline: 4.0659x; 4.0659x over previous
import functools

import jax
import jax.numpy as jnp
from jax import lax
from jax.experimental import pallas as pl
from jax.experimental.pallas import tpu as pltpu
from jax.experimental.pallas import tpu_sc as plsc

_VOCAB = 1000000
_D = 64
_B = 16384
_K = 20

_NC = 2
_NS = 16
_NW = _NC * _NS
_BPW = _B // _NW
_G = 32
_NG = _BPW // _G
_GK = _G * _K
_NEG_CHUNK = 128
_NJ = _GK // _NEG_CHUNK


def _sc_body(tgt_hbm, ctx_hbm, negf_hbm, u_hbm, v_hbm, w_hbm,
             pos_hbm, negdot_hbm, pred_hbm,
             idx_u, idx_v, idx_neg,
             u_a, v_a, n_a, u_b, v_b, n_b, w_vmem,
             pos_buf, pred_buf, neg_buf, sem_a, sem_b, sem_w):
    wid = lax.axis_index("s") * _NC + lax.axis_index("c")
    eb0 = wid * _BPW

    pltpu.async_copy(w_hbm, w_vmem, sem_w)
    pltpu.sync_copy(tgt_hbm.at[pl.ds(eb0, _BPW)], idx_u)
    pltpu.sync_copy(ctx_hbm.at[pl.ds(eb0, _BPW)], idx_v)
    pltpu.sync_copy(negf_hbm.at[pl.ds(wid * (_BPW * _K // 128),
                                      _BPW * _K // 128)], idx_neg)
    pltpu.make_async_copy(w_hbm, w_vmem, sem_w).wait()

    def issue(g, ub, vb, nb, sem):
        pltpu.async_copy(u_hbm.at[idx_u.at[pl.ds(g * _G, _G)]], ub, sem)
        pltpu.async_copy(v_hbm.at[idx_v.at[pl.ds(g * _G, _G)]], vb, sem)
        for j in range(_NJ):
            pltpu.async_copy(
                v_hbm.at[idx_neg.at[g * _NJ + j]],
                nb.at[pl.ds(j * _NEG_CHUNK, _NEG_CHUNK)], sem)

    def wait_group(ub, vb, nb, sem):
        pltpu.make_async_copy(u_hbm.at[idx_u.at[pl.ds(0, _G)]], ub, sem).wait()
        pltpu.make_async_copy(v_hbm.at[idx_v.at[pl.ds(0, _G)]], vb, sem).wait()
        for j in range(_NJ):
            pltpu.make_async_copy(
                v_hbm.at[idx_neg.at[j]],
                nb.at[pl.ds(j * _NEG_CHUNK, _NEG_CHUNK)], sem).wait()

    def compute(g, ub, vb, nb):
        for blk in range(_G // 16):
            lanes = jnp.arange(16, dtype=jnp.int32) + blk * 16
            nrows = [lanes * _K + k for k in range(_K)]
            ob = g * _G + blk * 16
            nob = g * _GK + blk * 16 * _K

            init = [jnp.zeros((16,), jnp.float32)] * 7

            def dbody0(d, accs, _lr=lanes, _nr=nrows):
                dcol = jnp.full((16,), d, jnp.int32)
                u_d = plsc.load_gather(ub, [_lr, dcol])
                v_d = plsc.load_gather(vb, [_lr, dcol])
                w_d = plsc.load_gather(w_vmem, [dcol])
                out = [accs[0] + u_d * v_d, accs[1] + u_d * w_d]
                for k in range(5):
                    n_d = plsc.load_gather(nb, [_nr[k], dcol])
                    out.append(accs[2 + k] + u_d * n_d)
                return out

            accs = pl.loop(0, _D, init_carry=init, unroll=2)(dbody0)
            pos_buf[pl.ds(ob, 16)] = accs[0]
            pred_buf[pl.ds(ob, 16)] = accs[1]
            for k in range(5):
                neg_buf[pl.ds(nob + k * 16, 16)] = accs[2 + k]

            for kc in range(5, _K, 5):
                init = [jnp.zeros((16,), jnp.float32)] * 5

                def dbodyk(d, accs, _lr=lanes, _nr=nrows, _kc=kc):
                    dcol = jnp.full((16,), d, jnp.int32)
                    u_d = plsc.load_gather(ub, [_lr, dcol])
                    out = []
                    for k in range(5):
                        n_d = plsc.load_gather(nb, [_nr[_kc + k], dcol])
                        out.append(accs[k] + u_d * n_d)
                    return out

                accs = pl.loop(0, _D, init_carry=init, unroll=2)(dbodyk)
                for k in range(5):
                    neg_buf[pl.ds(nob + (kc + k) * 16, 16)] = accs[k]

    issue(0, u_a, v_a, n_a, sem_a)

    @pl.loop(0, _NG // 2)
    def _pair(p):
        g0 = 2 * p
        issue(g0 + 1, u_b, v_b, n_b, sem_b)
        wait_group(u_a, v_a, n_a, sem_a)
        compute(g0, u_a, v_a, n_a)

        @pl.when(p < _NG // 2 - 1)
        def _():
            issue(g0 + 2, u_a, v_a, n_a, sem_a)

        wait_group(u_b, v_b, n_b, sem_b)
        compute(g0 + 1, u_b, v_b, n_b)

    pltpu.sync_copy(pos_buf, pos_hbm.at[pl.ds(eb0, _BPW)])
    pltpu.sync_copy(pred_buf, pred_hbm.at[pl.ds(eb0, _BPW)])
    pltpu.sync_copy(neg_buf, negdot_hbm.at[pl.ds(eb0 * _K, _BPW * _K)])


_sc_dots = functools.partial(
    pl.kernel,
    out_type=[
        jax.ShapeDtypeStruct((_B,), jnp.float32),
        jax.ShapeDtypeStruct((_B * _K,), jnp.float32),
        jax.ShapeDtypeStruct((_B,), jnp.float32),
    ],
    mesh=plsc.VectorSubcoreMesh(
        core_axis_name="c", subcore_axis_name="s",
        num_cores=_NC, num_subcores=_NS),
    compiler_params=pltpu.CompilerParams(
        needs_layout_passes=False, use_tc_tiling_on_sc=False),
    scratch_types=[
        pltpu.VMEM((_BPW,), jnp.int32),
        pltpu.VMEM((_BPW,), jnp.int32),
        pltpu.VMEM((_BPW * _K // 128, 128), jnp.int32),
        pltpu.VMEM((_G, _D), jnp.float32),
        pltpu.VMEM((_G, _D), jnp.float32),
        pltpu.VMEM((_GK, _D), jnp.float32),
        pltpu.VMEM((_G, _D), jnp.float32),
        pltpu.VMEM((_G, _D), jnp.float32),
        pltpu.VMEM((_GK, _D), jnp.float32),
        pltpu.VMEM((_D,), jnp.float32),
        pltpu.VMEM((_BPW,), jnp.float32),
        pltpu.VMEM((_BPW,), jnp.float32),
        pltpu.VMEM((_BPW * _K,), jnp.float32),
        pltpu.SemaphoreType.DMA,
        pltpu.SemaphoreType.DMA,
        pltpu.SemaphoreType.DMA,
    ],
)(_sc_body)


def _tc_body(pos_ref, neg_ref, pred_ref, b_ref, loss_ref, fix_ref):
    pos = jnp.clip(pos_ref[...], -10.0, 10.0)
    neg = jnp.clip(neg_ref[...], -10.0, 10.0)
    sp_pos = jnp.maximum(-pos, 0.0) + jnp.log(1.0 + jnp.exp(-jnp.abs(pos)))
    sp_neg = jnp.maximum(neg, 0.0) + jnp.log(1.0 + jnp.exp(-jnp.abs(neg)))
    total = jnp.sum(sp_pos) + jnp.sum(sp_neg)
    loss_ref[0, 0] = total / _B
    fix_ref[...] = pred_ref[...] + b_ref[0, 0]


def _tc_finish(pos2d, neg2d, pred2d, b2d):
    return pl.pallas_call(
        _tc_body,
        out_shape=[
            jax.ShapeDtypeStruct((1, 1), jnp.float32),
            jax.ShapeDtypeStruct((_B // 128, 128), jnp.float32),
        ],
        in_specs=[
            pl.BlockSpec(memory_space=pltpu.VMEM),
            pl.BlockSpec(memory_space=pltpu.VMEM),
            pl.BlockSpec(memory_space=pltpu.VMEM),
            pl.BlockSpec(memory_space=pltpu.SMEM),
        ],
        out_specs=[
            pl.BlockSpec(memory_space=pltpu.SMEM),
            pl.BlockSpec(memory_space=pltpu.VMEM),
        ],
    )(pos2d, neg2d, pred2d, b2d)


def kernel(target_word, context_words, neg_words, u_table, v_table, W_dur, b_dur):
    tgt = target_word.astype(jnp.int32)
    ctx = context_words.astype(jnp.int32)
    negf = neg_words.astype(jnp.int32).reshape(_B * _K // 128, 128)
    w = W_dur.reshape(_D)
    pos, negdot, pred = _sc_dots(tgt, ctx, negf, u_table, v_table, w)
    loss, fix = _tc_finish(
        pos.reshape(_B // 128, 128),
        negdot.reshape(_B * _K // 128, 128),
        pred.reshape(_B // 128, 128),
        b_dur.reshape(1, 1),
    )
    return loss.reshape(()), fix.reshape(_B)


# speedup vs baseline: 5.4930x; 1.3510x over previous
import functools

import jax
import jax.numpy as jnp
from jax import lax
from jax.experimental import pallas as pl
from jax.experimental.pallas import tpu as pltpu
from jax.experimental.pallas import tpu_sc as plsc

_VOCAB = 1000000
_D = 64
_B = 16384
_K = 20

_NC = 2
_NS = 16
_NW = _NC * _NS
_BPW = _B // _NW
_G = 32
_NG = _BPW // _G
_GK = _G * _K
_NEG_CHUNK = 128
_NJ = _GK // _NEG_CHUNK


def _sc_body(tgt_hbm, ctx_hbm, negf_hbm, u_hbm, v_hbm, w_hbm,
             pos_hbm, negdot_hbm, pred_hbm,
             idx_u, idx_v, idx_neg,
             u_a, v_a, n_a, u_b, v_b, n_b, w_vmem,
             pos_buf, pred_buf, neg_buf, sem_a, sem_b, sem_w):
    wid = lax.axis_index("s") * _NC + lax.axis_index("c")
    eb0 = wid * _BPW

    pltpu.async_copy(w_hbm, w_vmem, sem_w)
    pltpu.sync_copy(tgt_hbm.at[pl.ds(eb0, _BPW)], idx_u)
    pltpu.sync_copy(ctx_hbm.at[pl.ds(eb0, _BPW)], idx_v)
    pltpu.sync_copy(negf_hbm.at[pl.ds(wid * (_BPW * _K // 128),
                                      _BPW * _K // 128)], idx_neg)
    pltpu.make_async_copy(w_hbm, w_vmem, sem_w).wait()

    def issue(g, ub, vb, nb, sem):
        pltpu.async_copy(u_hbm.at[idx_u.at[pl.ds(g * _G, _G)]], ub, sem)
        pltpu.async_copy(v_hbm.at[idx_v.at[pl.ds(g * _G, _G)]], vb, sem)
        for j in range(_NJ):
            pltpu.async_copy(
                v_hbm.at[idx_neg.at[g * _NJ + j]],
                nb.at[pl.ds(j * _NEG_CHUNK, _NEG_CHUNK)], sem)

    def wait_group(ub, vb, nb, sem):
        pltpu.make_async_copy(u_hbm.at[idx_u.at[pl.ds(0, _G)]], ub, sem).wait()
        pltpu.make_async_copy(v_hbm.at[idx_v.at[pl.ds(0, _G)]], vb, sem).wait()
        for j in range(_NJ):
            pltpu.make_async_copy(
                v_hbm.at[idx_neg.at[j]],
                nb.at[pl.ds(j * _NEG_CHUNK, _NEG_CHUNK)], sem).wait()

    def compute(g, ub, vb, nb):
        for blk in range(_G // 16):
            iota = jnp.arange(16, dtype=jnp.int32)
            lanes = iota + blk * 16
            nrows = [lanes * _K + k for k in range(_K)]
            ob = g * _G + blk * 16
            nob = g * _GK + blk * 16 * _K

            init = [jnp.zeros((16,), jnp.float32)] * 7

            def dbody0(d, accs, _lr=lanes, _nr=nrows):
                dcol = (iota + d) & (_D - 1)
                u_d = plsc.load_gather(ub, [_lr, dcol])
                v_d = plsc.load_gather(vb, [_lr, dcol])
                w_d = plsc.load_gather(w_vmem, [dcol])
                out = [accs[0] + u_d * v_d, accs[1] + u_d * w_d]
                for k in range(5):
                    n_d = plsc.load_gather(nb, [_nr[k], dcol])
                    out.append(accs[2 + k] + u_d * n_d)
                return out

            accs = pl.loop(0, _D, init_carry=init, unroll=2)(dbody0)
            pos_buf[pl.ds(ob, 16)] = accs[0]
            pred_buf[pl.ds(ob, 16)] = accs[1]
            for k in range(5):
                neg_buf[pl.ds(nob + k * 16, 16)] = accs[2 + k]

            for kc in range(5, _K, 5):
                init = [jnp.zeros((16,), jnp.float32)] * 5

                def dbodyk(d, accs, _lr=lanes, _nr=nrows, _kc=kc):
                    dcol = (iota + d) & (_D - 1)
                    u_d = plsc.load_gather(ub, [_lr, dcol])
                    out = []
                    for k in range(5):
                        n_d = plsc.load_gather(nb, [_nr[_kc + k], dcol])
                        out.append(accs[k] + u_d * n_d)
                    return out

                accs = pl.loop(0, _D, init_carry=init, unroll=2)(dbodyk)
                for k in range(5):
                    neg_buf[pl.ds(nob + (kc + k) * 16, 16)] = accs[k]

    issue(0, u_a, v_a, n_a, sem_a)

    @pl.loop(0, _NG // 2)
    def _pair(p):
        g0 = 2 * p
        issue(g0 + 1, u_b, v_b, n_b, sem_b)
        wait_group(u_a, v_a, n_a, sem_a)
        compute(g0, u_a, v_a, n_a)

        @pl.when(p < _NG // 2 - 1)
        def _():
            issue(g0 + 2, u_a, v_a, n_a, sem_a)

        wait_group(u_b, v_b, n_b, sem_b)
        compute(g0 + 1, u_b, v_b, n_b)

    pltpu.sync_copy(pos_buf, pos_hbm.at[pl.ds(eb0, _BPW)])
    pltpu.sync_copy(pred_buf, pred_hbm.at[pl.ds(eb0, _BPW)])
    pltpu.sync_copy(neg_buf, negdot_hbm.at[pl.ds(eb0 * _K, _BPW * _K)])


_sc_dots = functools.partial(
    pl.kernel,
    out_type=[
        jax.ShapeDtypeStruct((_B,), jnp.float32),
        jax.ShapeDtypeStruct((_B * _K,), jnp.float32),
        jax.ShapeDtypeStruct((_B,), jnp.float32),
    ],
    mesh=plsc.VectorSubcoreMesh(
        core_axis_name="c", subcore_axis_name="s",
        num_cores=_NC, num_subcores=_NS),
    compiler_params=pltpu.CompilerParams(
        needs_layout_passes=False, use_tc_tiling_on_sc=False),
    scratch_types=[
        pltpu.VMEM((_BPW,), jnp.int32),
        pltpu.VMEM((_BPW,), jnp.int32),
        pltpu.VMEM((_BPW * _K // 128, 128), jnp.int32),
        pltpu.VMEM((_G, _D), jnp.float32),
        pltpu.VMEM((_G, _D), jnp.float32),
        pltpu.VMEM((_GK, _D), jnp.float32),
        pltpu.VMEM((_G, _D), jnp.float32),
        pltpu.VMEM((_G, _D), jnp.float32),
        pltpu.VMEM((_GK, _D), jnp.float32),
        pltpu.VMEM((_D,), jnp.float32),
        pltpu.VMEM((_BPW,), jnp.float32),
        pltpu.VMEM((_BPW,), jnp.float32),
        pltpu.VMEM((_BPW * _K,), jnp.float32),
        pltpu.SemaphoreType.DMA,
        pltpu.SemaphoreType.DMA,
        pltpu.SemaphoreType.DMA,
    ],
)(_sc_body)


def _tc_body(pos_ref, neg_ref, pred_ref, b_ref, loss_ref, fix_ref):
    pos = jnp.clip(pos_ref[...], -10.0, 10.0)
    neg = jnp.clip(neg_ref[...], -10.0, 10.0)
    sp_pos = jnp.maximum(-pos, 0.0) + jnp.log(1.0 + jnp.exp(-jnp.abs(pos)))
    sp_neg = jnp.maximum(neg, 0.0) + jnp.log(1.0 + jnp.exp(-jnp.abs(neg)))
    total = jnp.sum(sp_pos) + jnp.sum(sp_neg)
    loss_ref[0, 0] = total / _B
    fix_ref[...] = pred_ref[...] + b_ref[0, 0]


def _tc_finish(pos2d, neg2d, pred2d, b2d):
    return pl.pallas_call(
        _tc_body,
        out_shape=[
            jax.ShapeDtypeStruct((1, 1), jnp.float32),
            jax.ShapeDtypeStruct((_B // 128, 128), jnp.float32),
        ],
        in_specs=[
            pl.BlockSpec(memory_space=pltpu.VMEM),
            pl.BlockSpec(memory_space=pltpu.VMEM),
            pl.BlockSpec(memory_space=pltpu.VMEM),
            pl.BlockSpec(memory_space=pltpu.SMEM),
        ],
        out_specs=[
            pl.BlockSpec(memory_space=pltpu.SMEM),
            pl.BlockSpec(memory_space=pltpu.VMEM),
        ],
    )(pos2d, neg2d, pred2d, b2d)


def kernel(target_word, context_words, neg_words, u_table, v_table, W_dur, b_dur):
    tgt = target_word.astype(jnp.int32)
    ctx = context_words.astype(jnp.int32)
    negf = neg_words.astype(jnp.int32).reshape(_B * _K // 128, 128)
    w = W_dur.reshape(_D)
    pos, negdot, pred = _sc_dots(tgt, ctx, negf, u_table, v_table, w)
    loss, fix = _tc_finish(
        pos.reshape(_B // 128, 128),
        negdot.reshape(_B * _K // 128, 128),
        pred.reshape(_B // 128, 128),
        b_dur.reshape(1, 1),
    )
    return loss.reshape(()), fix.reshape(_B)
